# baseline (speedup 1.0000x reference)
import functools

import jax
import jax.numpy as jnp
from jax import lax
from jax.experimental import pallas as pl
from jax.experimental.pallas import tpu as pltpu
from jax.experimental.pallas import tpu_sc as plsc

B = 4
NUM_CLASSES = 80
NCOLS = 4 + NUM_CLASSES
IMG = 512.0
NC = 2
NS = 16
NW = NC * NS
L = 16

_LEVELS = (
    (8.0, 64, 64, 0, 256, 32, 2),
    (16.0, 32, 32, 4096, 128, 32, 1),
    (32.0, 16, 16, 5120, 32, 32, 1),
    (64.0, 8, 8, 5376, 16, 16, 1),
    (128.0, 4, 4, 5440, 16, 4, 1),
)
_TOTAL_ROWS = 5456

_AUX_OFF = []
_off = 0
for _s, _w, _h, _ro, _K, _nw, _rd in _LEVELS:
    _hw = _h * _w
    _AUX_OFF.append((_off, _off + B * _hw))
    _off += B * _hw + B * 4 * _hw
_AUX_LEN = _off


def _scratch_types():
    d = {}
    for i, (_, w, h, _, K, _, _) in enumerate(_LEVELS):
        R = K // w
        d[f"vcls{i}"] = pltpu.VMEM((R, w, NUM_CLASSES), jnp.float32)
        d[f"vcnt{i}"] = pltpu.VMEM((K,), jnp.float32)
        d[f"vreg{i}"] = pltpu.VMEM((4 * K,), jnp.float32)
        d[f"vout{i}"] = pltpu.VMEM((K, 128), jnp.float32)
        d[f"insem{i}"] = pltpu.SemaphoreType.DMA
    d["csbuf"] = pltpu.VMEM((L,), jnp.float32)
    d["outsem"] = pltpu.SemaphoreType.DMA
    return d


@functools.cache
def _build_sc_decode():
    mesh = plsc.VectorSubcoreMesh(
        core_axis_name="c", subcore_axis_name="s", num_cores=NC, num_subcores=NS
    )
    return functools.partial(
        pl.kernel,
        out_type=jax.ShapeDtypeStruct((B, _TOTAL_ROWS, 128), jnp.float32),
        mesh=mesh,
        scratch_types=_scratch_types(),
        compiler_params=pltpu.CompilerParams(
            use_tc_tiling_on_sc=True, needs_layout_passes=False
        ),
    )(_sc_decode)


def _sc_decode(clsT3, clsT4, clsT5, clsT6, clsT7, aux, out, **scr):
    wid = lax.axis_index("c") * NS + lax.axis_index("s")
    lane = lax.iota(jnp.int32, L)
    cls_refs = (clsT3, clsT4, clsT5, clsT6, clsT7)
    csbuf = scr["csbuf"]

    def chunk_coords(i, ci):
        _, w, h, _, K, _, _ = _LEVELS[i]
        R = K // w
        cpb = h // R
        return ci // cpb, (ci % cpb) * R

    def in_copies(i, ci):
        stride, w, h, row_off, K, nw, rd = _LEVELS[i]
        R = K // w
        hw = h * w
        b, h0 = chunk_coords(i, ci)
        c_off, r_off = _AUX_OFF[i]
        sem = scr[f"insem{i}"]
        cps = [
            pltpu.make_async_copy(
                cls_refs[i].at[b, pl.ds(h0, R), :, :], scr[f"vcls{i}"], sem
            ),
            pltpu.make_async_copy(
                aux.at[pl.ds(c_off + b * hw + h0 * w, K)], scr[f"vcnt{i}"], sem
            ),
        ]
        for k in range(4):
            cps.append(
                pltpu.make_async_copy(
                    aux.at[pl.ds(r_off + (b * 4 + k) * hw + h0 * w, K)],
                    scr[f"vreg{i}"].at[pl.ds(k * K, K)],
                    sem,
                )
            )
        return cps

    def out_copy(i, ci):
        _, w, h, row_off, K, _, _ = _LEVELS[i]
        cpb = h // (K // w)
        b = ci // cpb
        return pltpu.make_async_copy(
            scr[f"vout{i}"],
            out.at[b, pl.ds(row_off + (ci % cpb) * K, K), :],
            scr["outsem"],
        )

    def compute(i, ci):
        stride, w, h, row_off, K, nw, rd = _LEVELS[i]
        shift = w.bit_length() - 1
        b, h0 = chunk_coords(i, ci)
        for c in in_copies(i, ci):
            c.wait()
        vcls = scr[f"vcls{i}"]
        vcnt = scr[f"vcnt{i}"]
        vregr = scr[f"vreg{i}"]
        vout = scr[f"vout{i}"]

        def group(g, carry):
            j0 = g * L
            local = j0 + lane
            xs = ((local & (w - 1)).astype(jnp.float32) + 0.5) * stride
            ys = ((h0 + (local >> shift)).astype(jnp.float32) + 0.5) * stride
            dl = jnp.exp(vregr[pl.ds(j0, L)]) * stride
            dt = jnp.exp(vregr[pl.ds(K + j0, L)]) * stride
            dr = jnp.exp(vregr[pl.ds(2 * K + j0, L)]) * stride
            db = jnp.exp(vregr[pl.ds(3 * K + j0, L)]) * stride
            xmin = jnp.minimum(jnp.maximum(xs - dl, 0.0), IMG)
            ymin = jnp.minimum(jnp.maximum(ys - dt, 0.0), IMG)
            xmax = jnp.minimum(jnp.maximum(xs + dr, 0.0), IMG)
            ymax = jnp.minimum(jnp.maximum(ys + db, 0.0), IMG)
            zero = lane * 0
            plsc.store_scatter(vout, [local, zero], xmin)
            plsc.store_scatter(vout, [local, zero + 1], ymin)
            plsc.store_scatter(vout, [local, zero + 2], xmax)
            plsc.store_scatter(vout, [local, zero + 3], ymax)
            cs = 1.0 / (1.0 + jnp.exp(-vcnt[pl.ds(j0, L)]))
            csbuf[...] = cs

            @plsc.parallel_loop(0, L, unroll=2)
            def loc(j):
                csj = plsc.load_gather(csbuf, [jnp.full((L,), j, jnp.int32)])
                row = j0 + j
                rowv = jnp.full((L,), row, jnp.int32)
                rr = row >> shift
                x = row & (w - 1)
                for c0 in range(0, NUM_CLASSES, L):
                    v = vcls[rr, x, pl.ds(c0, L)]
                    s = csj / (1.0 + jnp.exp(-v))
                    plsc.store_scatter(vout, [rowv, lane + (4 + c0)], s)

            return carry

        lax.fori_loop(0, K // L, group, 0)
        out_copy(i, ci).start()

    def when_workers(nw, fn):
        if nw == NW:
            fn()
        else:
            pl.when(wid < nw)(fn)

    for i in range(5):
        def start_level(i=i):
            for c in in_copies(i, wid):
                c.start()

        when_workers(_LEVELS[i][5], start_level)

    compute(0, wid)
    for c in in_copies(0, wid + NW):
        c.start()
    for i in range(1, 5):
        when_workers(_LEVELS[i][5], lambda i=i: compute(i, wid))
    out_copy(0, wid).wait()
    compute(0, wid + NW)

    out_copy(0, wid + NW).wait()
    for i in range(1, 5):
        when_workers(_LEVELS[i][5], lambda i=i: out_copy(i, wid).wait())


def kernel(cls_p3, cntr_p3, regr_p3, cls_p4, cntr_p4, regr_p4,
           cls_p5, cntr_p5, regr_p5, cls_p6, cntr_p6, regr_p6,
           cls_p7, cntr_p7, regr_p7):
    clsT = [
        jnp.transpose(c, (0, 2, 3, 1))
        for c in (cls_p3, cls_p4, cls_p5, cls_p6, cls_p7)
    ]
    aux = jnp.concatenate([
        cntr_p3.reshape(-1), regr_p3.reshape(-1),
        cntr_p4.reshape(-1), regr_p4.reshape(-1),
        cntr_p5.reshape(-1), regr_p5.reshape(-1),
        cntr_p6.reshape(-1), regr_p6.reshape(-1),
        cntr_p7.reshape(-1), regr_p7.reshape(-1),
    ])
    out = _build_sc_decode()(*clsT, aux)
    return out[:, :, :NCOLS]


# speedup vs baseline: 1.1122x; 1.1122x over previous
import functools

import jax
import jax.numpy as jnp
from jax import lax
from jax.experimental import pallas as pl
from jax.experimental.pallas import tpu as pltpu
from jax.experimental.pallas import tpu_sc as plsc

B = 4
NUM_CLASSES = 80
NCOLS = 4 + NUM_CLASSES
IMG = 512.0
NC = 2
NS = 16
NW = NC * NS
L = 16

_LEVELS = (
    (8.0, 64, 64, 0, 128, 32, 4),
    (16.0, 32, 32, 4096, 128, 32, 1),
    (32.0, 16, 16, 5120, 32, 32, 1),
    (64.0, 8, 8, 5376, 16, 16, 1),
    (128.0, 4, 4, 5440, 16, 4, 1),
)
_TOTAL_ROWS = 5456

def _scratch_types():
    d = {}
    for i, (_, w, h, _, K, _, _) in enumerate(_LEVELS):
        R = K // w
        d[f"vcls{i}"] = pltpu.VMEM((R, w, NUM_CLASSES), jnp.float32)
        Hc = min(h, 8)
        d[f"vcnt{i}"] = pltpu.VMEM((Hc, w), jnp.float32)
        d[f"vreg{i}"] = pltpu.VMEM((4, Hc, w), jnp.float32)
        d[f"vout{i}"] = pltpu.VMEM((K, 128), jnp.float32)
        d[f"insem{i}"] = pltpu.SemaphoreType.DMA
    d["csbuf"] = pltpu.VMEM((L,), jnp.float32)
    d["outsem"] = pltpu.SemaphoreType.DMA
    return d


@functools.cache
def _build_sc_decode():
    mesh = plsc.VectorSubcoreMesh(
        core_axis_name="c", subcore_axis_name="s", num_cores=NC, num_subcores=NS
    )
    return functools.partial(
        pl.kernel,
        out_type=jax.ShapeDtypeStruct((B, _TOTAL_ROWS, 128), jnp.float32),
        mesh=mesh,
        scratch_types=_scratch_types(),
        compiler_params=pltpu.CompilerParams(
            use_tc_tiling_on_sc=True, needs_layout_passes=False
        ),
    )(_sc_decode)


def _sc_decode(clsT3, clsT4, clsT5, clsT6, clsT7,
               cntr3, cntr4, cntr5, cntr6, cntr7,
               regr3, regr4, regr5, regr6, regr7, out, **scr):
    wid = lax.axis_index("c") * NS + lax.axis_index("s")
    lane = lax.iota(jnp.int32, L)
    cls_refs = (clsT3, clsT4, clsT5, clsT6, clsT7)
    cnt_refs = (cntr3, cntr4, cntr5, cntr6, cntr7)
    reg_refs = (regr3, regr4, regr5, regr6, regr7)
    csbuf = scr["csbuf"]

    def chunk_coords(i, ci):
        _, w, h, _, K, _, _ = _LEVELS[i]
        R = K // w
        cpb = h // R
        return ci // cpb, (ci % cpb) * R

    def in_copies(i, ci):
        stride, w, h, row_off, K, nw, rd = _LEVELS[i]
        R = K // w
        Hc = min(h, 8)
        b, h0 = chunk_coords(i, ci)
        h8 = pl.multiple_of(h0 & ~7, 8)
        sem = scr[f"insem{i}"]
        return [
            pltpu.make_async_copy(
                cls_refs[i].at[b, pl.ds(h0, R), :, :], scr[f"vcls{i}"], sem
            ),
            pltpu.make_async_copy(
                cnt_refs[i].at[b, 0, pl.ds(h8, Hc), :], scr[f"vcnt{i}"], sem
            ),
            pltpu.make_async_copy(
                reg_refs[i].at[b, :, pl.ds(h8, Hc), :], scr[f"vreg{i}"], sem
            ),
        ]

    def out_copy(i, ci):
        _, w, h, row_off, K, _, _ = _LEVELS[i]
        cpb = h // (K // w)
        b = ci // cpb
        return pltpu.make_async_copy(
            scr[f"vout{i}"],
            out.at[b, pl.ds(row_off + (ci % cpb) * K, K), :],
            scr["outsem"],
        )

    def compute(i, ci):
        stride, w, h, row_off, K, nw, rd = _LEVELS[i]
        shift = w.bit_length() - 1
        b, h0 = chunk_coords(i, ci)
        hrel = h0 - (h0 & ~7)
        for c in in_copies(i, ci):
            c.wait()
        vcls = scr[f"vcls{i}"]
        vcnt = scr[f"vcnt{i}"]
        vregr = scr[f"vreg{i}"]
        vout = scr[f"vout{i}"]

        def group(g, carry):
            j0 = g * L
            local = j0 + lane
            xs = ((local & (w - 1)).astype(jnp.float32) + 0.5) * stride
            ys = ((h0 + (local >> shift)).astype(jnp.float32) + 0.5) * stride
            if w >= L:
                rr = hrel + (j0 >> shift)
                u = j0 & (w - 1)
                dl = jnp.exp(vregr[0, rr, pl.ds(u, L)]) * stride
                dt = jnp.exp(vregr[1, rr, pl.ds(u, L)]) * stride
                dr = jnp.exp(vregr[2, rr, pl.ds(u, L)]) * stride
                db = jnp.exp(vregr[3, rr, pl.ds(u, L)]) * stride
                cn = vcnt[rr, pl.ds(u, L)]
            else:
                ir = hrel + (local >> shift)
                iw = local & (w - 1)
                four = [
                    plsc.load_gather(
                        vregr, [jnp.full((L,), k, jnp.int32), ir, iw]
                    )
                    for k in range(4)
                ]
                dl, dt, dr, db = [jnp.exp(v) * stride for v in four]
                cn = plsc.load_gather(vcnt, [ir, iw])
            xmin = jnp.minimum(jnp.maximum(xs - dl, 0.0), IMG)
            ymin = jnp.minimum(jnp.maximum(ys - dt, 0.0), IMG)
            xmax = jnp.minimum(jnp.maximum(xs + dr, 0.0), IMG)
            ymax = jnp.minimum(jnp.maximum(ys + db, 0.0), IMG)
            zero = lane * 0
            plsc.store_scatter(vout, [local, zero], xmin)
            plsc.store_scatter(vout, [local, zero + 1], ymin)
            plsc.store_scatter(vout, [local, zero + 2], xmax)
            plsc.store_scatter(vout, [local, zero + 3], ymax)
            cs = 1.0 / (1.0 + jnp.exp(-cn))
            csbuf[...] = cs

            @plsc.parallel_loop(0, L, unroll=2)
            def loc(j):
                csj = plsc.load_gather(csbuf, [jnp.full((L,), j, jnp.int32)])
                row = j0 + j
                rowv = jnp.full((L,), row, jnp.int32)
                rr = row >> shift
                x = row & (w - 1)
                for c0 in range(0, NUM_CLASSES, L):
                    v = vcls[rr, x, pl.ds(c0, L)]
                    s = csj / (1.0 + jnp.exp(-v))
                    plsc.store_scatter(vout, [rowv, lane + (4 + c0)], s)

            return carry

        lax.fori_loop(0, K // L, group, 0)
        out_copy(i, ci).start()

    def when_workers(nw, fn):
        if nw == NW:
            fn()
        else:
            pl.when(wid < nw)(fn)

    for i in range(5):
        def start_level(i=i):
            for c in in_copies(i, wid):
                c.start()

        when_workers(_LEVELS[i][5], start_level)

    compute(0, wid)
    for c in in_copies(0, wid + NW):
        c.start()
    for i in range(1, 5):
        when_workers(_LEVELS[i][5], lambda i=i: compute(i, wid))
    for r in range(1, 4):
        out_copy(0, wid + (r - 1) * NW).wait()
        compute(0, wid + r * NW)
        if r < 3:
            for c in in_copies(0, wid + (r + 1) * NW):
                c.start()

    out_copy(0, wid + 3 * NW).wait()
    for i in range(1, 5):
        when_workers(_LEVELS[i][5], lambda i=i: out_copy(i, wid).wait())


def kernel(cls_p3, cntr_p3, regr_p3, cls_p4, cntr_p4, regr_p4,
           cls_p5, cntr_p5, regr_p5, cls_p6, cntr_p6, regr_p6,
           cls_p7, cntr_p7, regr_p7):
    clsT = [
        jnp.transpose(c, (0, 2, 3, 1))
        for c in (cls_p3, cls_p4, cls_p5, cls_p6, cls_p7)
    ]
    out = _build_sc_decode()(
        *clsT,
        cntr_p3, cntr_p4, cntr_p5, cntr_p6, cntr_p7,
        regr_p3, regr_p4, regr_p5, regr_p6, regr_p7,
    )
    return out[:, :, :NCOLS]


# speedup vs baseline: 1.1165x; 1.0038x over previous
import functools

import jax
import jax.numpy as jnp
from jax import lax
from jax.experimental import pallas as pl
from jax.experimental.pallas import tpu as pltpu
from jax.experimental.pallas import tpu_sc as plsc

B = 4
NUM_CLASSES = 80
NCOLS = 4 + NUM_CLASSES
IMG = 512.0
NC = 2
NS = 16
NW = NC * NS
L = 16

_LEVELS = (
    (8.0, 64, 64, 0, 128, 32, 4),
    (16.0, 32, 32, 4096, 128, 32, 1),
    (32.0, 16, 16, 5120, 32, 32, 1),
    (64.0, 8, 8, 5376, 16, 16, 1),
    (128.0, 4, 4, 5440, 16, 4, 1),
)
_TOTAL_ROWS = 5456

def _scratch_types():
    d = {}
    for i, (_, w, h, _, K, _, _) in enumerate(_LEVELS):
        R = K // w
        d[f"vcls{i}"] = pltpu.VMEM((R, w, NUM_CLASSES), jnp.float32)
        Hc = min(h, 8)
        d[f"vcnt{i}"] = pltpu.VMEM((Hc, w), jnp.float32)
        d[f"vreg{i}"] = pltpu.VMEM((4, Hc, w), jnp.float32)
        d[f"vout{i}"] = pltpu.VMEM((K, NCOLS), jnp.float32)
        d[f"insem{i}"] = pltpu.SemaphoreType.DMA
    d["csbuf"] = pltpu.VMEM((L,), jnp.float32)
    d["outsem"] = pltpu.SemaphoreType.DMA
    return d


@functools.cache
def _build_sc_decode():
    mesh = plsc.VectorSubcoreMesh(
        core_axis_name="c", subcore_axis_name="s", num_cores=NC, num_subcores=NS
    )
    return functools.partial(
        pl.kernel,
        out_type=jax.ShapeDtypeStruct((B, _TOTAL_ROWS, NCOLS), jnp.float32),
        mesh=mesh,
        scratch_types=_scratch_types(),
        compiler_params=pltpu.CompilerParams(
            use_tc_tiling_on_sc=True, needs_layout_passes=False
        ),
    )(_sc_decode)


def _sc_decode(clsT3, clsT4, clsT5, clsT6, clsT7,
               cntr3, cntr4, cntr5, cntr6, cntr7,
               regr3, regr4, regr5, regr6, regr7, out, **scr):
    wid = lax.axis_index("c") * NS + lax.axis_index("s")
    lane = lax.iota(jnp.int32, L)
    cls_refs = (clsT3, clsT4, clsT5, clsT6, clsT7)
    cnt_refs = (cntr3, cntr4, cntr5, cntr6, cntr7)
    reg_refs = (regr3, regr4, regr5, regr6, regr7)
    csbuf = scr["csbuf"]

    def chunk_coords(i, ci):
        _, w, h, _, K, _, _ = _LEVELS[i]
        R = K // w
        cpb = h // R
        return ci // cpb, (ci % cpb) * R

    def in_copies(i, ci):
        stride, w, h, row_off, K, nw, rd = _LEVELS[i]
        R = K // w
        Hc = min(h, 8)
        b, h0 = chunk_coords(i, ci)
        h8 = pl.multiple_of(h0 & ~7, 8)
        sem = scr[f"insem{i}"]
        return [
            pltpu.make_async_copy(
                cls_refs[i].at[b, pl.ds(h0, R), :, :], scr[f"vcls{i}"], sem
            ),
            pltpu.make_async_copy(
                cnt_refs[i].at[b, 0, pl.ds(h8, Hc), :], scr[f"vcnt{i}"], sem
            ),
            pltpu.make_async_copy(
                reg_refs[i].at[b, :, pl.ds(h8, Hc), :], scr[f"vreg{i}"], sem
            ),
        ]

    def out_copy(i, ci):
        _, w, h, row_off, K, _, _ = _LEVELS[i]
        cpb = h // (K // w)
        b = ci // cpb
        return pltpu.make_async_copy(
            scr[f"vout{i}"],
            out.at[b, pl.ds(row_off + (ci % cpb) * K, K), :],
            scr["outsem"],
        )

    def compute(i, ci):
        stride, w, h, row_off, K, nw, rd = _LEVELS[i]
        shift = w.bit_length() - 1
        b, h0 = chunk_coords(i, ci)
        hrel = h0 - (h0 & ~7)
        for c in in_copies(i, ci):
            c.wait()
        vcls = scr[f"vcls{i}"]
        vcnt = scr[f"vcnt{i}"]
        vregr = scr[f"vreg{i}"]
        vout = scr[f"vout{i}"]

        def group(g, carry):
            j0 = g * L
            local = j0 + lane
            xs = ((local & (w - 1)).astype(jnp.float32) + 0.5) * stride
            ys = ((h0 + (local >> shift)).astype(jnp.float32) + 0.5) * stride
            if w >= L:
                rr = hrel + (j0 >> shift)
                u = j0 & (w - 1)
                dl = jnp.exp(vregr[0, rr, pl.ds(u, L)]) * stride
                dt = jnp.exp(vregr[1, rr, pl.ds(u, L)]) * stride
                dr = jnp.exp(vregr[2, rr, pl.ds(u, L)]) * stride
                db = jnp.exp(vregr[3, rr, pl.ds(u, L)]) * stride
                cn = vcnt[rr, pl.ds(u, L)]
            else:
                ir = hrel + (local >> shift)
                iw = local & (w - 1)
                four = [
                    plsc.load_gather(
                        vregr, [jnp.full((L,), k, jnp.int32), ir, iw]
                    )
                    for k in range(4)
                ]
                dl, dt, dr, db = [jnp.exp(v) * stride for v in four]
                cn = plsc.load_gather(vcnt, [ir, iw])
            xmin = jnp.minimum(jnp.maximum(xs - dl, 0.0), IMG)
            ymin = jnp.minimum(jnp.maximum(ys - dt, 0.0), IMG)
            xmax = jnp.minimum(jnp.maximum(xs + dr, 0.0), IMG)
            ymax = jnp.minimum(jnp.maximum(ys + db, 0.0), IMG)
            zero = lane * 0
            plsc.store_scatter(vout, [local, zero], xmin)
            plsc.store_scatter(vout, [local, zero + 1], ymin)
            plsc.store_scatter(vout, [local, zero + 2], xmax)
            plsc.store_scatter(vout, [local, zero + 3], ymax)
            cs = 1.0 / (1.0 + jnp.exp(-cn))
            csbuf[...] = cs

            @plsc.parallel_loop(0, L, unroll=2)
            def loc(j):
                csj = plsc.load_gather(csbuf, [jnp.full((L,), j, jnp.int32)])
                row = j0 + j
                rowv = jnp.full((L,), row, jnp.int32)
                rr = row >> shift
                x = row & (w - 1)
                for c0 in range(0, NUM_CLASSES, L):
                    v = vcls[rr, x, pl.ds(c0, L)]
                    s = csj / (1.0 + jnp.exp(-v))
                    plsc.store_scatter(vout, [rowv, lane + (4 + c0)], s)

            return carry

        lax.fori_loop(0, K // L, group, 0)
        out_copy(i, ci).start()

    def when_workers(nw, fn):
        if nw == NW:
            fn()
        else:
            pl.when(wid < nw)(fn)

    for i in range(5):
        def start_level(i=i):
            for c in in_copies(i, wid):
                c.start()

        when_workers(_LEVELS[i][5], start_level)

    compute(0, wid)
    for c in in_copies(0, wid + NW):
        c.start()
    for i in range(1, 5):
        when_workers(_LEVELS[i][5], lambda i=i: compute(i, wid))
    for r in range(1, 4):
        out_copy(0, wid + (r - 1) * NW).wait()
        compute(0, wid + r * NW)
        if r < 3:
            for c in in_copies(0, wid + (r + 1) * NW):
                c.start()

    out_copy(0, wid + 3 * NW).wait()
    for i in range(1, 5):
        when_workers(_LEVELS[i][5], lambda i=i: out_copy(i, wid).wait())


def kernel(cls_p3, cntr_p3, regr_p3, cls_p4, cntr_p4, regr_p4,
           cls_p5, cntr_p5, regr_p5, cls_p6, cntr_p6, regr_p6,
           cls_p7, cntr_p7, regr_p7):
    clsT = [
        jnp.transpose(c, (0, 2, 3, 1))
        for c in (cls_p3, cls_p4, cls_p5, cls_p6, cls_p7)
    ]
    out = _build_sc_decode()(
        *clsT,
        cntr_p3, cntr_p4, cntr_p5, cntr_p6, cntr_p7,
        regr_p3, regr_p4, regr_p5, regr_p6, regr_p7,
    )
    return out


# speedup vs baseline: 1.4024x; 1.2561x over previous
import functools

import jax
import jax.numpy as jnp
from jax import lax
from jax.experimental import pallas as pl
from jax.experimental.pallas import tpu as pltpu
from jax.experimental.pallas import tpu_sc as plsc
import jax.experimental.layout as xl

B = 4
NUM_CLASSES = 80
NCOLS = 4 + NUM_CLASSES
IMG = 512.0
NC = 2
NS = 16
NW = NC * NS
L = 16

_LEVELS = (
    (8.0, 64, 64, 0, 128, 32, 4),
    (16.0, 32, 32, 4096, 128, 32, 1),
    (32.0, 16, 16, 5120, 32, 32, 1),
    (64.0, 8, 8, 5376, 16, 16, 1),
    (128.0, 4, 4, 5440, 16, 4, 1),
)
_TOTAL_ROWS = 5456

def _scratch_types():
    d = {}
    for i, (_, w, h, _, K, _, _) in enumerate(_LEVELS):
        R = K // w
        d[f"vcls{i}"] = pltpu.VMEM((R, w, NUM_CLASSES), jnp.float32)
        Hc = min(h, 8)
        d[f"vcnt{i}"] = pltpu.VMEM((Hc, w), jnp.float32)
        d[f"vreg{i}"] = pltpu.VMEM((4, Hc, w), jnp.float32)
        d[f"vout{i}"] = pltpu.VMEM((K, NCOLS), jnp.float32)
        d[f"insem{i}"] = pltpu.SemaphoreType.DMA
    d["csbuf"] = pltpu.VMEM((L,), jnp.float32)
    d["outsem"] = pltpu.SemaphoreType.DMA
    return d


@functools.cache
def _build_sc_decode():
    mesh = plsc.VectorSubcoreMesh(
        core_axis_name="c", subcore_axis_name="s", num_cores=NC, num_subcores=NS
    )
    return functools.partial(
        pl.kernel,
        out_type=jax.ShapeDtypeStruct((B, _TOTAL_ROWS, NCOLS), jnp.float32),
        mesh=mesh,
        scratch_types=_scratch_types(),
        compiler_params=pltpu.CompilerParams(
            use_tc_tiling_on_sc=True, needs_layout_passes=False
        ),
    )(_sc_decode)


def _sc_decode(clsT3, clsT4, clsT5, clsT6, clsT7,
               cntr3, cntr4, cntr5, cntr6, cntr7,
               regr3, regr4, regr5, regr6, regr7, out, **scr):
    wid = lax.axis_index("c") * NS + lax.axis_index("s")
    lane = lax.iota(jnp.int32, L)
    cls_refs = (clsT3, clsT4, clsT5, clsT6, clsT7)
    cnt_refs = (cntr3, cntr4, cntr5, cntr6, cntr7)
    reg_refs = (regr3, regr4, regr5, regr6, regr7)
    csbuf = scr["csbuf"]

    def chunk_coords(i, ci):
        _, w, h, _, K, _, _ = _LEVELS[i]
        R = K // w
        cpb = h // R
        return ci // cpb, (ci % cpb) * R

    def in_copies(i, ci):
        stride, w, h, row_off, K, nw, rd = _LEVELS[i]
        R = K // w
        Hc = min(h, 8)
        b, h0 = chunk_coords(i, ci)
        h8 = pl.multiple_of(h0 & ~7, 8)
        sem = scr[f"insem{i}"]
        return [
            pltpu.make_async_copy(
                cls_refs[i].at[b, pl.ds(h0, R), :, :], scr[f"vcls{i}"], sem
            ),
            pltpu.make_async_copy(
                cnt_refs[i].at[b, 0, pl.ds(h8, Hc), :], scr[f"vcnt{i}"], sem
            ),
            pltpu.make_async_copy(
                reg_refs[i].at[b, :, pl.ds(h8, Hc), :], scr[f"vreg{i}"], sem
            ),
        ]

    def out_copy(i, ci):
        _, w, h, row_off, K, _, _ = _LEVELS[i]
        cpb = h // (K // w)
        b = ci // cpb
        return pltpu.make_async_copy(
            scr[f"vout{i}"],
            out.at[b, pl.ds(row_off + (ci % cpb) * K, K), :],
            scr["outsem"],
        )

    def compute(i, ci):
        stride, w, h, row_off, K, nw, rd = _LEVELS[i]
        shift = w.bit_length() - 1
        b, h0 = chunk_coords(i, ci)
        hrel = h0 - (h0 & ~7)
        for c in in_copies(i, ci):
            c.wait()
        vcls = scr[f"vcls{i}"]
        vcnt = scr[f"vcnt{i}"]
        vregr = scr[f"vreg{i}"]
        vout = scr[f"vout{i}"]

        def group(g, carry):
            j0 = g * L
            local = j0 + lane
            xs = ((local & (w - 1)).astype(jnp.float32) + 0.5) * stride
            ys = ((h0 + (local >> shift)).astype(jnp.float32) + 0.5) * stride
            if w >= L:
                rr = hrel + (j0 >> shift)
                u = j0 & (w - 1)
                dl = jnp.exp(vregr[0, rr, pl.ds(u, L)]) * stride
                dt = jnp.exp(vregr[1, rr, pl.ds(u, L)]) * stride
                dr = jnp.exp(vregr[2, rr, pl.ds(u, L)]) * stride
                db = jnp.exp(vregr[3, rr, pl.ds(u, L)]) * stride
                cn = vcnt[rr, pl.ds(u, L)]
            else:
                ir = hrel + (local >> shift)
                iw = local & (w - 1)
                four = [
                    plsc.load_gather(
                        vregr, [jnp.full((L,), k, jnp.int32), ir, iw]
                    )
                    for k in range(4)
                ]
                dl, dt, dr, db = [jnp.exp(v) * stride for v in four]
                cn = plsc.load_gather(vcnt, [ir, iw])
            xmin = jnp.minimum(jnp.maximum(xs - dl, 0.0), IMG)
            ymin = jnp.minimum(jnp.maximum(ys - dt, 0.0), IMG)
            xmax = jnp.minimum(jnp.maximum(xs + dr, 0.0), IMG)
            ymax = jnp.minimum(jnp.maximum(ys + db, 0.0), IMG)
            zero = lane * 0
            plsc.store_scatter(vout, [local, zero], xmin)
            plsc.store_scatter(vout, [local, zero + 1], ymin)
            plsc.store_scatter(vout, [local, zero + 2], xmax)
            plsc.store_scatter(vout, [local, zero + 3], ymax)
            cs = 1.0 / (1.0 + jnp.exp(-cn))
            csbuf[...] = cs

            @plsc.parallel_loop(0, L, unroll=2)
            def loc(j):
                csj = plsc.load_gather(csbuf, [jnp.full((L,), j, jnp.int32)])
                row = j0 + j
                rowv = jnp.full((L,), row, jnp.int32)
                rr = row >> shift
                x = row & (w - 1)
                for c0 in range(0, NUM_CLASSES, L):
                    v = vcls[rr, x, pl.ds(c0, L)]
                    s = csj / (1.0 + jnp.exp(-v))
                    plsc.store_scatter(vout, [rowv, lane + (4 + c0)], s)

            return carry

        lax.fori_loop(0, K // L, group, 0)
        out_copy(i, ci).start()

    def when_workers(nw, fn):
        if nw == NW:
            fn()
        else:
            pl.when(wid < nw)(fn)

    for i in range(5):
        def start_level(i=i):
            for c in in_copies(i, wid):
                c.start()

        when_workers(_LEVELS[i][5], start_level)

    compute(0, wid)
    for c in in_copies(0, wid + NW):
        c.start()
    for i in range(1, 5):
        when_workers(_LEVELS[i][5], lambda i=i: compute(i, wid))
    for r in range(1, 4):
        out_copy(0, wid + (r - 1) * NW).wait()
        compute(0, wid + r * NW)
        if r < 3:
            for c in in_copies(0, wid + (r + 1) * NW):
                c.start()

    out_copy(0, wid + 3 * NW).wait()
    for i in range(1, 5):
        when_workers(_LEVELS[i][5], lambda i=i: out_copy(i, wid).wait())


def kernel(cls_p3, cntr_p3, regr_p3, cls_p4, cntr_p4, regr_p4,
           cls_p5, cntr_p5, regr_p5, cls_p6, cntr_p6, regr_p6,
           cls_p7, cntr_p7, regr_p7):
    clsT = [
        jnp.transpose(c, (0, 2, 3, 1))
        for c in (cls_p3, cls_p4, cls_p5, cls_p6, cls_p7)
    ]
    out = _build_sc_decode()(
        *clsT,
        cntr_p3, cntr_p4, cntr_p5, cntr_p6, cntr_p7,
        regr_p3, regr_p4, regr_p5, regr_p6, regr_p7,
    )
    return xl.with_layout_constraint(out, xl.Layout((0, 1, 2), tiling=()))
